# Initial kernel scaffold; baseline (speedup 1.0000x reference)
import jax
import jax.numpy as jnp
from jax.experimental import pallas as pl


def kernel(x, edge_index, W_gcn, b_gcn, W_lin, b_lin):
    raise NotImplementedError("write your pallas kernel here")


# speedup vs baseline: 19.1815x; 19.1815x over previous
import functools

import jax
import jax.numpy as jnp
from jax import lax
from jax.experimental import pallas as pl
from jax.experimental.pallas import tpu as pltpu
from jax.experimental.pallas import tpu_sc as plsc

N = 10000
E = 320000
D = 128
A = 64
NC = 2
NS = 16
NW = NC * NS
K = 128
NCH = E // K
CPT = 160
NCHP = CPT * NS
EPAD = (NCHP - NCH) * K
NH = 5000
NHP = 5120
RPT = NHP // NS
CH = 160
NPD = NS * 640


def _deg_body(dst_hbm, out_hbm, di_v, ones_v, zeros_v, deg_sh):
    c = lax.axis_index("c")
    s = lax.axis_index("s")
    for j in range(K // 16):
        ones_v[pl.ds(j * 16, 16)] = jnp.full((16,), 1.0, jnp.float32)
    for j in range(640 // 16):
        zeros_v[pl.ds(j * 16, 16)] = jnp.zeros((16,), jnp.float32)
    pltpu.sync_copy(zeros_v, deg_sh.at[pl.ds(640 * s, 640)])
    pltpu.sync_copy(dst_hbm.at[s, pl.ds(c * (CPT // NC), CPT // NC)], di_v)
    plsc.subcore_barrier()

    def body(i, _):
        pltpu.sync_copy(ones_v, deg_sh.at[di_v.at[i]], add=True)
        return 0
    lax.fori_loop(0, CPT // NC, body, 0)
    plsc.subcore_barrier()
    @pl.when(s < NS - 1)
    def _():
        pltpu.sync_copy(deg_sh.at[pl.ds(640 * s, 640)], zeros_v)
        pltpu.sync_copy(zeros_v, out_hbm.at[pl.ds(c * N + 640 * s, 640)])
    @pl.when(s == NS - 1)
    def _():
        pltpu.sync_copy(deg_sh.at[pl.ds(9600, 400)], zeros_v.at[pl.ds(0, 400)])
        pltpu.sync_copy(zeros_v.at[pl.ds(0, 400)],
                        out_hbm.at[pl.ds(c * N + 9600, 400)])


def _prep_body(pT_ref, x_ref, wg_ref, wl_ref, z_ref, dinv_ref):
    deg = 1.0 + pT_ref[:, 0:1] + pT_ref[:, 1:2]
    dinv = lax.rsqrt(deg)
    y = jnp.dot(x_ref[:], wg_ref[:], preferred_element_type=jnp.float32)
    y = jnp.dot(y, wl_ref[:], preferred_element_type=jnp.float32)
    z_ref[:, 0:A] = dinv * y
    z_ref[:, A:2 * A] = jnp.zeros_like(y)
    dinv_ref[:] = dinv


def _agg_body(src_hbm, dst_hbm, z_hbm, out_hbm,
              si_v, di_v, rows_v, zrows_v, bnc_v, acc_sh):
    c = lax.axis_index("c")
    s = lax.axis_index("s")
    for r in range(32):
        for j in range(2 * A // 16):
            zrows_v[r, pl.ds(j * 16, 16)] = jnp.zeros((16,), jnp.float32)

    def zbody(i, _):
        pltpu.sync_copy(zrows_v, acc_sh.at[pl.ds(RPT * s + 32 * i, 32)])
        return 0
    lax.fori_loop(0, RPT // 32, zbody, 0)
    pltpu.sync_copy(src_hbm.at[s], si_v)
    pltpu.sync_copy(dst_hbm.at[s], di_v)
    base = c * NH

    def clamp(i, _):
        for j in range(K // 16):
            d = di_v[i, pl.ds(j * 16, 16)]
            rel = d - base
            ok = (rel >= 0) & (rel < NH)
            di_v[i, pl.ds(j * 16, 16)] = jnp.where(
                ok, rel, NH + (d & 63))
        return 0
    lax.fori_loop(0, CPT, clamp, 0)
    plsc.subcore_barrier()

    def body(i, _):
        pltpu.sync_copy(z_hbm.at[si_v.at[i]], rows_v)
        pltpu.sync_copy(rows_v, acc_sh.at[di_v.at[i]], add=True)
        return 0
    lax.fori_loop(0, CPT, body, 0)
    plsc.subcore_barrier()

    nfull = jnp.where(s == NS - 1, 1, 2)

    def wout(i, _):
        pltpu.sync_copy(acc_sh.at[pl.ds(RPT * s + CH * i, CH)], bnc_v)
        pltpu.sync_copy(bnc_v, out_hbm.at[pl.ds(c * NH + RPT * s + CH * i, CH)])
        return 0
    lax.fori_loop(0, nfull, wout, 0)
    @pl.when(s == NS - 1)
    def _():
        pltpu.sync_copy(acc_sh.at[pl.ds(4960, 40)], bnc_v.at[pl.ds(0, 40)])
        pltpu.sync_copy(bnc_v.at[pl.ds(0, 40)],
                        out_hbm.at[pl.ds(c * NH + 4960, 40)])


def _fin_body(agg_ref, z_ref, dinv_ref, bg_ref, wl_ref, bl_ref, o_ref):
    ssum = agg_ref[:, 0:A] + z_ref[:, 0:A]
    bias = jnp.dot(bg_ref[:], wl_ref[:],
                   preferred_element_type=jnp.float32) + bl_ref[:]
    o_ref[:] = dinv_ref[:] * ssum + bias


@functools.lru_cache(maxsize=1)
def _sc_kernels():
    mesh = plsc.VectorSubcoreMesh(
        core_axis_name="c", subcore_axis_name="s",
        num_cores=NC, num_subcores=NS)
    deg_kernel = pl.kernel(
        _deg_body,
        out_type=jax.ShapeDtypeStruct((NC * N,), jnp.float32),
        mesh=mesh,
        scratch_types=[
            pltpu.VMEM((CPT // NC, K), jnp.int32),
            pltpu.VMEM((K,), jnp.float32),
            pltpu.VMEM((640,), jnp.float32),
            pltpu.VMEM_SHARED((NPD,), jnp.float32),
        ])
    agg_kernel = pl.kernel(
        _agg_body,
        out_type=jax.ShapeDtypeStruct((N, 2 * A), jnp.float32),
        mesh=mesh,
        scratch_types=[
            pltpu.VMEM((CPT, K), jnp.int32),
            pltpu.VMEM((CPT, K), jnp.int32),
            pltpu.VMEM((K, 2 * A), jnp.float32),
            pltpu.VMEM((32, 2 * A), jnp.float32),
            pltpu.VMEM((CH, 2 * A), jnp.float32),
            pltpu.VMEM_SHARED((NHP, 2 * A), jnp.float32),
        ])
    return deg_kernel, agg_kernel


def kernel(x, edge_index, W_gcn, b_gcn, W_lin, b_lin):
    pad = jnp.arange(EPAD, dtype=jnp.int32) % (NPD - N)
    src2 = (jnp.concatenate([edge_index[0], pad])
            .reshape(CPT, NS, K).transpose(1, 0, 2))
    dst2 = (jnp.concatenate([edge_index[1], N + pad])
            .reshape(CPT, NS, K).transpose(1, 0, 2))
    _deg_kernel, _agg_kernel = _sc_kernels()

    deg_p = _deg_kernel(dst2)
    deg_pT = deg_p.reshape(NC, N).T

    B = 2000
    z, dinv = pl.pallas_call(
        _prep_body,
        grid=(N // B,),
        in_specs=[
            pl.BlockSpec((B, NC), lambda i: (i, 0)),
            pl.BlockSpec((B, D), lambda i: (i, 0)),
            pl.BlockSpec((D, D), lambda i: (0, 0)),
            pl.BlockSpec((D, A), lambda i: (0, 0)),
        ],
        out_specs=[
            pl.BlockSpec((B, 2 * A), lambda i: (i, 0)),
            pl.BlockSpec((B, 1), lambda i: (i, 0)),
        ],
        out_shape=[
            jax.ShapeDtypeStruct((N, 2 * A), jnp.float32),
            jax.ShapeDtypeStruct((N, 1), jnp.float32),
        ],
    )(deg_pT, x, W_gcn, W_lin)

    agg = _agg_kernel(src2, dst2, z)

    logits = pl.pallas_call(
        _fin_body,
        grid=(N // B,),
        in_specs=[
            pl.BlockSpec((B, 2 * A), lambda i: (i, 0)),
            pl.BlockSpec((B, 2 * A), lambda i: (i, 0)),
            pl.BlockSpec((B, 1), lambda i: (i, 0)),
            pl.BlockSpec((1, D), lambda i: (0, 0)),
            pl.BlockSpec((D, A), lambda i: (0, 0)),
            pl.BlockSpec((1, A), lambda i: (0, 0)),
        ],
        out_specs=pl.BlockSpec((B, A), lambda i: (i, 0)),
        out_shape=jax.ShapeDtypeStruct((N, A), jnp.float32),
    )(agg, z, dinv, b_gcn.reshape(1, D), W_lin, b_lin.reshape(1, A))
    return logits


# speedup vs baseline: 30.0668x; 1.5675x over previous
import functools

import jax
import jax.numpy as jnp
from jax import lax
from jax.experimental import pallas as pl
from jax.experimental.pallas import tpu as pltpu
from jax.experimental.pallas import tpu_sc as plsc

N = 10000
E = 320000
D = 128
A = 64
NC = 2
NS = 16
NW = NC * NS
K = 128
NCH = E // K
CPT = 160
CPC = CPT // NC
NCHP = CPT * NS
EPAD = (NCHP - NCH) * K
NH = 5000
NHP = 5120
RPT = NHP // NS
CH = 160
NPD = NS * 640


def _deg_body(dst_hbm, out_hbm, di_v, ones_v, zeros_v, deg_sh):
    c = lax.axis_index("c")
    s = lax.axis_index("s")
    for j in range(K // 16):
        ones_v[pl.ds(j * 16, 16)] = jnp.full((16,), 1.0, jnp.float32)
    for j in range(640 // 16):
        zeros_v[pl.ds(j * 16, 16)] = jnp.zeros((16,), jnp.float32)
    pltpu.sync_copy(zeros_v, deg_sh.at[pl.ds(640 * s, 640)])
    pltpu.sync_copy(dst_hbm.at[s, pl.ds(c * (CPT // NC), CPT // NC)], di_v)
    plsc.subcore_barrier()

    def body(i, _):
        pltpu.sync_copy(ones_v, deg_sh.at[di_v.at[i]], add=True)
        return 0
    lax.fori_loop(0, CPT // NC, body, 0)
    plsc.subcore_barrier()
    @pl.when(s < NS - 1)
    def _():
        pltpu.sync_copy(deg_sh.at[pl.ds(640 * s, 640)], zeros_v)
        pltpu.sync_copy(zeros_v, out_hbm.at[pl.ds(c * N + 640 * s, 640)])
    @pl.when(s == NS - 1)
    def _():
        pltpu.sync_copy(deg_sh.at[pl.ds(9600, 400)], zeros_v.at[pl.ds(0, 400)])
        pltpu.sync_copy(zeros_v.at[pl.ds(0, 400)],
                        out_hbm.at[pl.ds(c * N + 9600, 400)])


def _prep_body(pT_ref, x_ref, wg_ref, wl_ref, z_ref, dinv_ref):
    h = pl.program_id(1)
    deg = 1.0 + pT_ref[:, 0:1] + pT_ref[:, 1:2]
    dinv = lax.rsqrt(deg)
    y = jnp.dot(x_ref[:], wg_ref[:], preferred_element_type=jnp.float32)
    y = jnp.dot(y, wl_ref[:], preferred_element_type=jnp.float32)
    zv = dinv * y
    zz = jnp.zeros_like(y)

    @pl.when(h == 0)
    def _():
        z_ref[:, 0:A] = zv
        z_ref[:, A:2 * A] = zz

    @pl.when(h == 1)
    def _():
        z_ref[:, 0:A] = zz
        z_ref[:, A:2 * A] = zv

    dinv_ref[:] = dinv


def _agg_body(src_hbm, dst_hbm, z_hbm, out_hbm,
              si_v, di_v, rows_v, zrows_v, bnc_v, acc_sh):
    c = lax.axis_index("c")
    s = lax.axis_index("s")
    for r in range(32):
        for j in range(2 * A // 16):
            zrows_v[r, pl.ds(j * 16, 16)] = jnp.zeros((16,), jnp.float32)

    def zbody(i, _):
        pltpu.sync_copy(zrows_v, acc_sh.at[pl.ds(RPT * s + 32 * i, 32)])
        return 0
    lax.fori_loop(0, RPT // 32, zbody, 0)
    pltpu.sync_copy(src_hbm.at[s, pl.ds(c * CPC, CPC)], si_v)
    pltpu.sync_copy(dst_hbm.at[s, pl.ds(c * CPC, CPC)], di_v)

    def remap(i, _):
        for j in range(K // 16):
            d = di_v[i, pl.ds(j * 16, 16)]
            g = si_v[i, pl.ds(j * 16, 16)] + (d & 1) * N
            si_v[i, pl.ds(j * 16, 16)] = g
            di_v[i, pl.ds(j * 16, 16)] = lax.shift_right_logical(d, 1)
        return 0
    lax.fori_loop(0, CPC, remap, 0)
    plsc.subcore_barrier()

    def body(i, _):
        pltpu.sync_copy(z_hbm.at[si_v.at[i]], rows_v)
        pltpu.sync_copy(rows_v, acc_sh.at[di_v.at[i]], add=True)
        return 0
    lax.fori_loop(0, CPC, body, 0)
    plsc.subcore_barrier()

    nfull = jnp.where(s == NS - 1, 1, 2)

    def wout(i, _):
        pltpu.sync_copy(acc_sh.at[pl.ds(RPT * s + CH * i, CH)], bnc_v)
        pltpu.sync_copy(bnc_v, out_hbm.at[pl.ds(c * NH + RPT * s + CH * i, CH)])
        return 0
    lax.fori_loop(0, nfull, wout, 0)
    @pl.when(s == NS - 1)
    def _():
        pltpu.sync_copy(acc_sh.at[pl.ds(4960, 40)], bnc_v.at[pl.ds(0, 40)])
        pltpu.sync_copy(bnc_v.at[pl.ds(0, 40)],
                        out_hbm.at[pl.ds(c * NH + 4960, 40)])


def _fin_body(agg_ref, z_ref, dinv_ref, bg_ref, wl_ref, bl_ref, o_ref):
    ssum = agg_ref[0] + agg_ref[1] + z_ref[:, 0:A]
    bias = jnp.dot(bg_ref[:], wl_ref[:],
                   preferred_element_type=jnp.float32) + bl_ref[:]
    o_ref[:] = dinv_ref[:] * ssum + bias


@functools.lru_cache(maxsize=1)
def _sc_kernels():
    mesh = plsc.VectorSubcoreMesh(
        core_axis_name="c", subcore_axis_name="s",
        num_cores=NC, num_subcores=NS)
    deg_kernel = pl.kernel(
        _deg_body,
        out_type=jax.ShapeDtypeStruct((NC * N,), jnp.float32),
        mesh=mesh,
        scratch_types=[
            pltpu.VMEM((CPT // NC, K), jnp.int32),
            pltpu.VMEM((K,), jnp.float32),
            pltpu.VMEM((640,), jnp.float32),
            pltpu.VMEM_SHARED((NPD,), jnp.float32),
        ])
    agg_kernel = pl.kernel(
        _agg_body,
        out_type=jax.ShapeDtypeStruct((N, 2 * A), jnp.float32),
        mesh=mesh,
        scratch_types=[
            pltpu.VMEM((CPC, K), jnp.int32),
            pltpu.VMEM((CPC, K), jnp.int32),
            pltpu.VMEM((K, 2 * A), jnp.float32),
            pltpu.VMEM((32, 2 * A), jnp.float32),
            pltpu.VMEM((CH, 2 * A), jnp.float32),
            pltpu.VMEM_SHARED((NHP, 2 * A), jnp.float32),
        ])
    return deg_kernel, agg_kernel


def kernel(x, edge_index, W_gcn, b_gcn, W_lin, b_lin):
    pad = jnp.arange(EPAD, dtype=jnp.int32) % (NPD - N)
    src2 = (jnp.concatenate([edge_index[0], pad])
            .reshape(CPT, NS, K).transpose(1, 0, 2))
    dst2 = (jnp.concatenate([edge_index[1], N + pad])
            .reshape(CPT, NS, K).transpose(1, 0, 2))
    _deg_kernel, _agg_kernel = _sc_kernels()

    deg_p = _deg_kernel(dst2)
    deg_pT = deg_p.reshape(NC, N).T

    B = 2000
    G = N // B
    z2, dinv = pl.pallas_call(
        _prep_body,
        grid=(G, 2),
        in_specs=[
            pl.BlockSpec((B, NC), lambda i, h: (i, 0)),
            pl.BlockSpec((B, D), lambda i, h: (i, 0)),
            pl.BlockSpec((D, D), lambda i, h: (0, 0)),
            pl.BlockSpec((D, A), lambda i, h: (0, 0)),
        ],
        out_specs=[
            pl.BlockSpec((B, 2 * A), lambda i, h: (h * G + i, 0)),
            pl.BlockSpec((B, 1), lambda i, h: (i, 0)),
        ],
        out_shape=[
            jax.ShapeDtypeStruct((2 * N, 2 * A), jnp.float32),
            jax.ShapeDtypeStruct((N, 1), jnp.float32),
        ],
    )(deg_pT, x, W_gcn, W_lin)

    agg = _agg_kernel(src2, dst2, z2)
    aggp = agg.reshape(NC, 2 * NH, A)

    logits = pl.pallas_call(
        _fin_body,
        grid=(G,),
        in_specs=[
            pl.BlockSpec((NC, B, A), lambda i: (0, i, 0)),
            pl.BlockSpec((B, 2 * A), lambda i: (i, 0)),
            pl.BlockSpec((B, 1), lambda i: (i, 0)),
            pl.BlockSpec((1, D), lambda i: (0, 0)),
            pl.BlockSpec((D, A), lambda i: (0, 0)),
            pl.BlockSpec((1, A), lambda i: (0, 0)),
        ],
        out_specs=pl.BlockSpec((B, A), lambda i: (i, 0)),
        out_shape=jax.ShapeDtypeStruct((N, A), jnp.float32),
    )(aggp, z2, dinv, b_gcn.reshape(1, D), W_lin, b_lin.reshape(1, A))
    return logits


# speedup vs baseline: 41.0434x; 1.3651x over previous
import functools

import jax
import jax.numpy as jnp
from jax import lax
from jax.experimental import pallas as pl
from jax.experimental.pallas import tpu as pltpu
from jax.experimental.pallas import tpu_sc as plsc

N = 10000
E = 320000
D = 128
A = 64
NC = 2
NS = 16
NW = NC * NS
K = 128
NCH = E // K
CPT = 160
CPC = CPT // NC
NCHP = CPT * NS
EPAD = (NCHP - NCH) * K
NH = 5000
NHP = 5120
RPT = NHP // NS
CH = 160
NPD = NS * 640


def _deg_body(dst_hbm, out_hbm, di_v, ones_v, zeros_v, deg_sh):
    c = lax.axis_index("c")
    s = lax.axis_index("s")
    for j in range(K // 16):
        ones_v[pl.ds(j * 16, 16)] = jnp.full((16,), 1.0, jnp.float32)
    for j in range(640 // 16):
        zeros_v[pl.ds(j * 16, 16)] = jnp.zeros((16,), jnp.float32)
    pltpu.sync_copy(zeros_v, deg_sh.at[pl.ds(640 * s, 640)])
    pltpu.sync_copy(dst_hbm.at[s, pl.ds(c * (CPT // NC), CPT // NC)], di_v)
    plsc.subcore_barrier()

    def body(i, _):
        pltpu.sync_copy(ones_v, deg_sh.at[di_v.at[i]], add=True)
        return 0
    lax.fori_loop(0, CPT // NC, body, 0)
    plsc.subcore_barrier()
    @pl.when(s < NS - 1)
    def _():
        pltpu.sync_copy(deg_sh.at[pl.ds(640 * s, 640)], zeros_v)
        pltpu.sync_copy(zeros_v, out_hbm.at[pl.ds(c * N + 640 * s, 640)])
    @pl.when(s == NS - 1)
    def _():
        pltpu.sync_copy(deg_sh.at[pl.ds(9600, 400)], zeros_v.at[pl.ds(0, 400)])
        pltpu.sync_copy(zeros_v.at[pl.ds(0, 400)],
                        out_hbm.at[pl.ds(c * N + 9600, 400)])


def _prep_body(pT_ref, x_ref, wg_ref, wl_ref, z_ref, dinv_ref):
    h = pl.program_id(1)
    deg = 1.0 + pT_ref[:, 0:1] + pT_ref[:, 1:2]
    dinv = lax.rsqrt(deg)
    y = jnp.dot(x_ref[:], wg_ref[:], preferred_element_type=jnp.float32)
    y = jnp.dot(y, wl_ref[:], preferred_element_type=jnp.float32)
    zv = dinv * y
    zz = jnp.zeros_like(y)

    @pl.when(h == 0)
    def _():
        z_ref[:, 0:A] = zv
        z_ref[:, A:2 * A] = zz

    @pl.when(h == 1)
    def _():
        z_ref[:, 0:A] = zz
        z_ref[:, A:2 * A] = zv

    dinv_ref[:] = dinv


def _agg_body(src_hbm, dst_hbm, z_hbm, out_hbm,
              si_v, di_v, r0_v, r1_v, zrows_v, bnc_v, g0_sem, g1_sem,
              acc_sh):
    c = lax.axis_index("c")
    s = lax.axis_index("s")
    for r in range(32):
        for j in range(2 * A // 16):
            zrows_v[r, pl.ds(j * 16, 16)] = jnp.zeros((16,), jnp.float32)

    def zbody(i, _):
        pltpu.sync_copy(zrows_v, acc_sh.at[pl.ds(RPT * s + 32 * i, 32)])
        return 0
    lax.fori_loop(0, RPT // 32, zbody, 0)
    pltpu.sync_copy(src_hbm.at[s, pl.ds(c * CPC, CPC)], si_v)
    pltpu.sync_copy(dst_hbm.at[s, pl.ds(c * CPC, CPC)], di_v)

    def remap(i, _):
        for j in range(K // 16):
            d = di_v[i, pl.ds(j * 16, 16)]
            g = si_v[i, pl.ds(j * 16, 16)] + (d & 1) * N
            si_v[i, pl.ds(j * 16, 16)] = g
            di_v[i, pl.ds(j * 16, 16)] = lax.shift_right_logical(d, 1)
        return 0
    lax.fori_loop(0, CPC, remap, 0)
    plsc.subcore_barrier()

    pltpu.async_copy(z_hbm.at[si_v.at[0]], r0_v, g0_sem)
    pltpu.async_copy(z_hbm.at[si_v.at[1]], r1_v, g1_sem)

    def body(p, _):
        i = 2 * p
        pltpu.make_async_copy(z_hbm.at[si_v.at[i]], r0_v, g0_sem).wait()
        pltpu.sync_copy(r0_v, acc_sh.at[di_v.at[i]], add=True)

        @pl.when(i + 2 < CPC)
        def _():
            pltpu.async_copy(z_hbm.at[si_v.at[i + 2]], r0_v, g0_sem)

        pltpu.make_async_copy(z_hbm.at[si_v.at[i + 1]], r1_v, g1_sem).wait()
        pltpu.sync_copy(r1_v, acc_sh.at[di_v.at[i + 1]], add=True)

        @pl.when(i + 3 < CPC)
        def _():
            pltpu.async_copy(z_hbm.at[si_v.at[i + 3]], r1_v, g1_sem)
        return 0
    lax.fori_loop(0, CPC // 2, body, 0)
    plsc.subcore_barrier()

    nfull = jnp.where(s == NS - 1, 1, 2)

    def wout(i, _):
        pltpu.sync_copy(acc_sh.at[pl.ds(RPT * s + CH * i, CH)], bnc_v)
        pltpu.sync_copy(bnc_v, out_hbm.at[pl.ds(c * NH + RPT * s + CH * i, CH)])
        return 0
    lax.fori_loop(0, nfull, wout, 0)
    @pl.when(s == NS - 1)
    def _():
        pltpu.sync_copy(acc_sh.at[pl.ds(4960, 40)], bnc_v.at[pl.ds(0, 40)])
        pltpu.sync_copy(bnc_v.at[pl.ds(0, 40)],
                        out_hbm.at[pl.ds(c * NH + 4960, 40)])


def _fin_body(agg_ref, z_ref, dinv_ref, bg_ref, wl_ref, bl_ref, o_ref):
    ssum = agg_ref[0] + agg_ref[1] + z_ref[:, 0:A]
    bias = jnp.dot(bg_ref[:], wl_ref[:],
                   preferred_element_type=jnp.float32) + bl_ref[:]
    o_ref[:] = dinv_ref[:] * ssum + bias


@functools.lru_cache(maxsize=1)
def _sc_kernels():
    mesh = plsc.VectorSubcoreMesh(
        core_axis_name="c", subcore_axis_name="s",
        num_cores=NC, num_subcores=NS)
    deg_kernel = pl.kernel(
        _deg_body,
        out_type=jax.ShapeDtypeStruct((NC * N,), jnp.float32),
        mesh=mesh,
        scratch_types=[
            pltpu.VMEM((CPT // NC, K), jnp.int32),
            pltpu.VMEM((K,), jnp.float32),
            pltpu.VMEM((640,), jnp.float32),
            pltpu.VMEM_SHARED((NPD,), jnp.float32),
        ])
    agg_kernel = pl.kernel(
        _agg_body,
        out_type=jax.ShapeDtypeStruct((N, 2 * A), jnp.float32),
        mesh=mesh,
        scratch_types=[
            pltpu.VMEM((CPC, K), jnp.int32),
            pltpu.VMEM((CPC, K), jnp.int32),
            pltpu.VMEM((K, 2 * A), jnp.float32),
            pltpu.VMEM((K, 2 * A), jnp.float32),
            pltpu.VMEM((32, 2 * A), jnp.float32),
            pltpu.VMEM((CH, 2 * A), jnp.float32),
            pltpu.SemaphoreType.DMA,
            pltpu.SemaphoreType.DMA,
            pltpu.VMEM_SHARED((NHP, 2 * A), jnp.float32),
        ])
    return deg_kernel, agg_kernel


def kernel(x, edge_index, W_gcn, b_gcn, W_lin, b_lin):
    pad = jnp.arange(EPAD, dtype=jnp.int32) % (NPD - N)
    src2 = (jnp.concatenate([edge_index[0], pad])
            .reshape(CPT, NS, K).transpose(1, 0, 2))
    dst2 = (jnp.concatenate([edge_index[1], N + pad])
            .reshape(CPT, NS, K).transpose(1, 0, 2))
    _deg_kernel, _agg_kernel = _sc_kernels()

    deg_p = _deg_kernel(dst2)
    deg_pT = deg_p.reshape(NC, N).T

    B = 2000
    G = N // B
    z2, dinv = pl.pallas_call(
        _prep_body,
        grid=(G, 2),
        in_specs=[
            pl.BlockSpec((B, NC), lambda i, h: (i, 0)),
            pl.BlockSpec((B, D), lambda i, h: (i, 0)),
            pl.BlockSpec((D, D), lambda i, h: (0, 0)),
            pl.BlockSpec((D, A), lambda i, h: (0, 0)),
        ],
        out_specs=[
            pl.BlockSpec((B, 2 * A), lambda i, h: (h * G + i, 0)),
            pl.BlockSpec((B, 1), lambda i, h: (i, 0)),
        ],
        out_shape=[
            jax.ShapeDtypeStruct((2 * N, 2 * A), jnp.float32),
            jax.ShapeDtypeStruct((N, 1), jnp.float32),
        ],
    )(deg_pT, x, W_gcn, W_lin)

    agg = _agg_kernel(src2, dst2, z2)
    aggp = agg.reshape(NC, 2 * NH, A)

    logits = pl.pallas_call(
        _fin_body,
        grid=(G,),
        in_specs=[
            pl.BlockSpec((NC, B, A), lambda i: (0, i, 0)),
            pl.BlockSpec((B, 2 * A), lambda i: (i, 0)),
            pl.BlockSpec((B, 1), lambda i: (i, 0)),
            pl.BlockSpec((1, D), lambda i: (0, 0)),
            pl.BlockSpec((D, A), lambda i: (0, 0)),
            pl.BlockSpec((1, A), lambda i: (0, 0)),
        ],
        out_specs=pl.BlockSpec((B, A), lambda i: (i, 0)),
        out_shape=jax.ShapeDtypeStruct((N, A), jnp.float32),
    )(aggp, z2, dinv, b_gcn.reshape(1, D), W_lin, b_lin.reshape(1, A))
    return logits
